# baseline (speedup 1.0000x reference)
import jax
import jax.numpy as jnp
from jax import lax
from jax.experimental import pallas as pl
from jax.experimental.pallas import tpu as pltpu
from jax.experimental.pallas import tpu_sc as plsc

NC = 2
NS = 16
NW = NC * NS

NBUF = 8
LA = 4
LB = 2


def _make_sc_call(B, N, L, D):
    G = B * N
    gpw = G // NW
    wpb = NW // B
    npw = N // wpb
    assert gpw * NW == G and npw * wpb == N

    mesh = plsc.VectorSubcoreMesh(core_axis_name="c", subcore_axis_name="s")

    def body(x_hbm, idx_hbm, pe_hbm, out_hbm, idx_v, bufs, sem_in, sem_g, sem_out):
        wid = lax.axis_index("s") * NC + lax.axis_index("c")
        b = wid // wpb
        n0 = (wid % wpb) * npw
        pltpu.sync_copy(idx_hbm.at[b, pl.ds(n0, npw)], idx_v)

        h_in = [None] * gpw
        h_g = [None] * gpw
        h_out = [None] * gpw
        out_waited = [False] * gpw

        def s1(j):
            h_in[j] = pltpu.async_copy(
                x_hbm.at[b, n0 + j], bufs.at[j % NBUF], sem_in.at[j % NBUF])

        def s2(j):
            h_in[j].wait()
            h_g[j] = pltpu.async_copy(
                pe_hbm.at[idx_v.at[j]], bufs.at[j % NBUF], sem_g.at[j % NBUF],
                add=True)

        def s3(j):
            h_g[j].wait()
            h_out[j] = pltpu.async_copy(
                bufs.at[j % NBUF], out_hbm.at[b, n0 + j], sem_out.at[j % NBUF])

        for j in range(min(LA, gpw)):
            s1(j)
        for j in range(min(LB, gpw)):
            s2(j)
        for j in range(gpw):
            ja = j + LA
            if ja < gpw:
                if ja >= NBUF:
                    h_out[ja - NBUF].wait()
                    out_waited[ja - NBUF] = True
                s1(ja)
            jb = j + LB
            if jb < gpw:
                s2(jb)
            s3(j)
        for j in range(gpw):
            if not out_waited[j]:
                h_out[j].wait()

    return pl.kernel(
        body,
        out_type=jax.ShapeDtypeStruct((B, N, L, D), jnp.float32),
        mesh=mesh,
        scratch_types=[
            pltpu.VMEM((gpw, L), jnp.int32),
            pltpu.VMEM((NBUF, L, D), jnp.float32),
            pltpu.SemaphoreType.DMA((NBUF,)),
            pltpu.SemaphoreType.DMA((NBUF,)),
            pltpu.SemaphoreType.DMA((NBUF,)),
        ],
    )


def kernel(input_emb, position, pe):
    B, N, L, D = input_emb.shape
    idx = position.astype(jnp.int32)
    return _make_sc_call(B, N, L, D)(input_emb, idx, pe)
